# baseline (speedup 1.0000x reference)
import functools

import jax
import jax.numpy as jnp
from jax import lax
from jax.experimental import pallas as pl
from jax.experimental.pallas import tpu as pltpu
from jax.experimental.pallas import tpu_sc as plsc

VOCAB = 100000
EMBED_DIM = 64
BATCH = 4096
HIST = 50

TOTAL = BATCH * HIST
NUM_CORES = 2
NUM_SUBCORES = 16
NW = NUM_CORES * NUM_SUBCORES
PER_W = TOTAL // NW
CHUNK = 400
NCHUNK = PER_W // CHUNK
NBUF = 4
BATCH_PER_CHUNK = CHUNK // HIST
BATCH_PER_W = PER_W // HIST


def _make_kernel():
    mesh = plsc.VectorSubcoreMesh(
        core_axis_name="c", subcore_axis_name="s",
        num_cores=NUM_CORES, num_subcores=NUM_SUBCORES)

    @functools.partial(
        pl.kernel,
        out_type=jax.ShapeDtypeStruct((BATCH, HIST, EMBED_DIM), jnp.float32),
        mesh=mesh,
        compiler_params=pltpu.CompilerParams(use_tc_tiling_on_sc=False),
        scratch_types=(
            [pltpu.VMEM((CHUNK,), jnp.int32) for _ in range(NCHUNK)]
            + [pltpu.VMEM((CHUNK, EMBED_DIM), jnp.float32)
               for _ in range(NBUF)]
            + [pltpu.SemaphoreType.DMA for _ in range(2 * NBUF + 1)]
        ),
    )
    def gather_kernel(idx_hbm, table_hbm, out_hbm, *scratch):
        idx_refs = scratch[:NCHUNK]
        rows = scratch[NCHUNK:NCHUNK + NBUF]
        gsem = scratch[NCHUNK + NBUF:NCHUNK + 2 * NBUF]
        wsem = scratch[NCHUNK + 2 * NBUF:NCHUNK + 3 * NBUF]
        isem = scratch[NCHUNK + 3 * NBUF]
        wid = lax.axis_index("s") * NUM_CORES + lax.axis_index("c")
        batch0 = wid * BATCH_PER_W

        idx_copies = [
            pltpu.async_copy(idx_hbm.at[wid, c], idx_refs[c], isem)
            for c in range(NCHUNK)
        ]
        for cp in idx_copies:
            cp.wait()

        gathers = [None] * NCHUNK
        writes = [None] * NCHUNK

        def start_writes(c):
            b = c % NBUF
            return [
                pltpu.async_copy(
                    rows[b].at[pl.ds(j * HIST, HIST)],
                    out_hbm.at[batch0 + c * BATCH_PER_CHUNK + j],
                    wsem[b])
                for j in range(BATCH_PER_CHUNK)
            ]

        for c in range(NBUF):
            gathers[c] = pltpu.async_copy(
                table_hbm.at[idx_refs[c]], rows[c % NBUF], gsem[c % NBUF])
        for c in range(NCHUNK):
            nxt = c + NBUF - 1
            if c >= 1 and nxt < NCHUNK:
                b = (c - 1) % NBUF
                for w in writes[c - 1]:
                    w.wait()
                gathers[nxt] = pltpu.async_copy(
                    table_hbm.at[idx_refs[nxt]], rows[b], gsem[b])
            gathers[c].wait()
            writes[c] = start_writes(c)
        for c in range(max(0, NCHUNK - NBUF), NCHUNK):
            for w in writes[c]:
                w.wait()

    return gather_kernel


_gather = _make_kernel()


@jax.jit
def kernel(input_seq, embedding_matrix):
    idx = input_seq.reshape(NW, NCHUNK, CHUNK).astype(jnp.int32)
    return _gather(idx, embedding_matrix)


# speedup vs baseline: 1.0611x; 1.0611x over previous
import functools

import jax
import jax.numpy as jnp
from jax import lax
from jax.experimental import pallas as pl
from jax.experimental.pallas import tpu as pltpu
from jax.experimental.pallas import tpu_sc as plsc

VOCAB = 100000
EMBED_DIM = 64
BATCH = 4096
HIST = 50

NUM_CORES = 2
NUM_SUBCORES = 16
NW = NUM_CORES * NUM_SUBCORES
BLK = BATCH // NW
NCHUNK = HIST
NBUF = 4


def _make_kernel():
    mesh = plsc.VectorSubcoreMesh(
        core_axis_name="c", subcore_axis_name="s",
        num_cores=NUM_CORES, num_subcores=NUM_SUBCORES)

    @functools.partial(
        pl.kernel,
        out_type=jax.ShapeDtypeStruct((HIST, BATCH, EMBED_DIM), jnp.float32),
        mesh=mesh,
        compiler_params=pltpu.CompilerParams(use_tc_tiling_on_sc=False),
        scratch_types=(
            [pltpu.VMEM((NCHUNK, BLK), jnp.int32)]
            + [pltpu.VMEM((BLK, EMBED_DIM), jnp.float32)
               for _ in range(NBUF)]
            + [pltpu.SemaphoreType.DMA for _ in range(2 * NBUF + 1)]
        ),
    )
    def gather_kernel(idx_hbm, table_hbm, out_hbm, *scratch):
        idx_v = scratch[0]
        rows = scratch[1:1 + NBUF]
        gsem = scratch[1 + NBUF:1 + 2 * NBUF]
        wsem = scratch[1 + 2 * NBUF:1 + 3 * NBUF]
        isem = scratch[1 + 3 * NBUF]
        wid = lax.axis_index("s") * NUM_CORES + lax.axis_index("c")
        col0 = wid * BLK

        pltpu.async_copy(
            idx_hbm.at[:, pl.ds(col0, BLK)], idx_v, isem).wait()

        gathers = [None] * NCHUNK
        writes = [None] * NCHUNK

        for c in range(NBUF):
            gathers[c] = pltpu.async_copy(
                table_hbm.at[idx_v.at[c]], rows[c % NBUF], gsem[c % NBUF])
        for c in range(NCHUNK):
            nxt = c + NBUF - 1
            if c >= 1 and nxt < NCHUNK:
                b = (c - 1) % NBUF
                writes[c - 1].wait()
                gathers[nxt] = pltpu.async_copy(
                    table_hbm.at[idx_v.at[nxt]], rows[b], gsem[b])
            gathers[c].wait()
            writes[c] = pltpu.async_copy(
                rows[c % NBUF],
                out_hbm.at[c, pl.ds(col0, BLK)],
                wsem[c % NBUF])
        for c in range(max(0, NCHUNK - NBUF), NCHUNK):
            writes[c].wait()

    return gather_kernel


_gather = _make_kernel()


@jax.jit
def kernel(input_seq, embedding_matrix):
    idx_t = input_seq.T.astype(jnp.int32)
    out_t = _gather(idx_t, embedding_matrix)
    return out_t.transpose(1, 0, 2)
